# Initial kernel scaffold; baseline (speedup 1.0000x reference)
import jax
import jax.numpy as jnp
from jax.experimental import pallas as pl


def kernel(input_tokens, table):
    raise NotImplementedError("write your pallas kernel here")


# speedup vs baseline: 6.1154x; 6.1154x over previous
import functools

import jax
import jax.numpy as jnp
from jax import lax
from jax.experimental import pallas as pl
from jax.experimental.pallas import tpu as pltpu
from jax.experimental.pallas import tpu_sc as plsc

NUM_TOKENS = 100000
EMBED_DIM = 32
BATCH = 16384
HIST = 200

B = BATCH * HIST
NW = 32
B_PER_W = B // NW
CHUNK = 1024
N_CHUNKS = B_PER_W // CHUNK
IDX_ROWS = CHUNK // 128

assert B % (NW * CHUNK) == 0


def _make_kernel():
    mesh = plsc.VectorSubcoreMesh(core_axis_name="c", subcore_axis_name="s")

    @functools.partial(
        pl.kernel,
        out_type=jax.ShapeDtypeStruct((B, EMBED_DIM), jnp.float32),
        mesh=mesh,
        compiler_params=pltpu.CompilerParams(use_tc_tiling_on_sc=False),
        scratch_types=[
            pltpu.VMEM((IDX_ROWS, 128), jnp.int32),
            pltpu.VMEM((CHUNK, EMBED_DIM), jnp.float32),
            pltpu.SemaphoreType.DMA,
        ],
    )
    def gather_kernel(idx_hbm, table_hbm, out_hbm, idx_v, rows_v, sem):
        wid = lax.axis_index("s") * 2 + lax.axis_index("c")
        base = wid * B_PER_W

        def body(c, carry):
            off = pl.multiple_of(base + c * CHUNK, CHUNK)
            row_off = pl.multiple_of(
                wid * (B_PER_W // 128) + c * IDX_ROWS, IDX_ROWS)
            pltpu.sync_copy(idx_hbm.at[pl.ds(row_off, IDX_ROWS)], idx_v)
            copies = [
                pltpu.async_copy(
                    table_hbm.at[idx_v.at[r]],
                    rows_v.at[pl.ds(r * 128, 128)],
                    sem,
                )
                for r in range(IDX_ROWS)
            ]
            for cp in copies:
                cp.wait()
            pltpu.sync_copy(rows_v, out_hbm.at[pl.ds(off, CHUNK)])
            return carry

        lax.fori_loop(0, N_CHUNKS, body, 0)

    return gather_kernel


_GATHER = _make_kernel()


def kernel(input_tokens, table):
    idx = input_tokens.reshape(B // 128, 128).astype(jnp.int32)
    out = _GATHER(idx, table)
    return out.reshape(BATCH, HIST, EMBED_DIM)


# speedup vs baseline: 6.4685x; 1.0577x over previous
import functools

import jax
import jax.numpy as jnp
from jax import lax
from jax.experimental import pallas as pl
from jax.experimental.pallas import tpu as pltpu
from jax.experimental.pallas import tpu_sc as plsc

NUM_TOKENS = 100000
EMBED_DIM = 32
BATCH = 16384
HIST = 200

B = BATCH * HIST
NW = 32
B_PER_W = B // NW
CHUNK = 1024
N_CHUNKS = B_PER_W // CHUNK
IDX_ROWS = CHUNK // 128

assert B % (NW * CHUNK) == 0 and N_CHUNKS % 2 == 0 and N_CHUNKS >= 4


def _make_kernel():
    mesh = plsc.VectorSubcoreMesh(core_axis_name="c", subcore_axis_name="s")

    @functools.partial(
        pl.kernel,
        out_type=jax.ShapeDtypeStruct((B, EMBED_DIM), jnp.float32),
        mesh=mesh,
        compiler_params=pltpu.CompilerParams(use_tc_tiling_on_sc=False),
        scratch_types=[
            pltpu.VMEM((IDX_ROWS, 128), jnp.int32),
            pltpu.VMEM((IDX_ROWS, 128), jnp.int32),
            pltpu.VMEM((CHUNK, EMBED_DIM), jnp.float32),
            pltpu.VMEM((CHUNK, EMBED_DIM), jnp.float32),
            pltpu.SemaphoreType.DMA,
            pltpu.SemaphoreType.DMA,
            pltpu.SemaphoreType.DMA,
            pltpu.SemaphoreType.DMA,
            pltpu.SemaphoreType.DMA,
            pltpu.SemaphoreType.DMA,
        ],
    )
    def gather_kernel(idx_hbm, table_hbm, out_hbm,
                      idx0, idx1, rows0, rows1,
                      si0, si1, sg0, sg1, ss0, ss1):
        wid = lax.axis_index("s") * 2 + lax.axis_index("c")
        base = wid * B_PER_W
        idx_base = wid * (B_PER_W // 128)

        idx_bufs = (idx0, idx1)
        rows_bufs = (rows0, rows1)
        sems_i = (si0, si1)
        sems_g = (sg0, sg1)
        sems_s = (ss0, ss1)

        def issue_idx(c, b):
            row_off = pl.multiple_of(idx_base + c * IDX_ROWS, IDX_ROWS)
            pltpu.async_copy(
                idx_hbm.at[pl.ds(row_off, IDX_ROWS)], idx_bufs[b], sems_i[b])

        def wait_idx(b):
            pltpu.make_async_copy(
                idx_hbm.at[pl.ds(0, IDX_ROWS)], idx_bufs[b], sems_i[b]).wait()

        def issue_gathers(b):
            for r in range(IDX_ROWS):
                pltpu.async_copy(
                    table_hbm.at[idx_bufs[b].at[r]],
                    rows_bufs[b].at[pl.ds(r * 128, 128)],
                    sems_g[b])

        def wait_gathers(b):
            pltpu.make_async_copy(
                out_hbm.at[pl.ds(0, CHUNK)], rows_bufs[b], sems_g[b]).wait()

        def issue_store(c, b):
            off = pl.multiple_of(base + c * CHUNK, CHUNK)
            pltpu.async_copy(
                rows_bufs[b], out_hbm.at[pl.ds(off, CHUNK)], sems_s[b])

        def wait_store(b):
            pltpu.make_async_copy(
                rows_bufs[b], out_hbm.at[pl.ds(0, CHUNK)], sems_s[b]).wait()

        issue_idx(0, 0)
        issue_idx(1, 1)
        wait_idx(0)
        issue_gathers(0)

        wait_gathers(0)
        issue_idx(2, 0)
        issue_store(0, 0)
        wait_idx(1)
        issue_gathers(1)
        wait_gathers(1)
        issue_idx(3, 1)
        issue_store(1, 1)
        wait_store(0)
        wait_idx(0)
        issue_gathers(0)

        def body(i, carry):
            g = 2 * i
            wait_gathers(0)
            issue_idx(g + 2, 0)
            issue_store(g, 0)
            wait_store(1)
            wait_idx(1)
            issue_gathers(1)
            wait_gathers(1)
            issue_idx(g + 3, 1)
            issue_store(g + 1, 1)
            wait_store(0)
            wait_idx(0)
            issue_gathers(0)
            return carry

        lax.fori_loop(1, N_CHUNKS // 2 - 1, body, 0)

        wait_gathers(0)
        issue_store(N_CHUNKS - 2, 0)
        wait_store(1)
        wait_idx(1)
        issue_gathers(1)
        wait_gathers(1)
        issue_store(N_CHUNKS - 1, 1)
        wait_store(0)
        wait_store(1)

    return gather_kernel


_GATHER = _make_kernel()


def kernel(input_tokens, table):
    idx = input_tokens.reshape(B // 128, 128).astype(jnp.int32)
    out = _GATHER(idx, table)
    return out.reshape(BATCH, HIST, EMBED_DIM)


# speedup vs baseline: 6.4788x; 1.0016x over previous
import functools

import jax
import jax.numpy as jnp
from jax import lax
from jax.experimental import pallas as pl
from jax.experimental.pallas import tpu as pltpu
from jax.experimental.pallas import tpu_sc as plsc

NUM_TOKENS = 100000
EMBED_DIM = 32
BATCH = 16384
HIST = 200

B = BATCH * HIST
NW = 32
B_PER_W = B // NW
CHUNK = 1024
N_CHUNKS = B_PER_W // CHUNK
IDX_ROWS = CHUNK // 128

assert B % (NW * CHUNK) == 0 and N_CHUNKS % 2 == 0 and N_CHUNKS >= 4


def _make_kernel():
    mesh = plsc.VectorSubcoreMesh(core_axis_name="c", subcore_axis_name="s")

    @functools.partial(
        pl.kernel,
        out_type=jax.ShapeDtypeStruct((B, EMBED_DIM), jnp.float32),
        mesh=mesh,
        compiler_params=pltpu.CompilerParams(use_tc_tiling_on_sc=False),
        scratch_types=[
            pltpu.VMEM((CHUNK,), jnp.int32),
            pltpu.VMEM((CHUNK,), jnp.int32),
            pltpu.VMEM((CHUNK, EMBED_DIM), jnp.float32),
            pltpu.VMEM((CHUNK, EMBED_DIM), jnp.float32),
            pltpu.SemaphoreType.DMA,
            pltpu.SemaphoreType.DMA,
            pltpu.SemaphoreType.DMA,
            pltpu.SemaphoreType.DMA,
            pltpu.SemaphoreType.DMA,
            pltpu.SemaphoreType.DMA,
        ],
    )
    def gather_kernel(idx_hbm, table_hbm, out_hbm,
                      idx0, idx1, rows0, rows1,
                      si0, si1, sg0, sg1, ss0, ss1):
        wid = lax.axis_index("s") * 2 + lax.axis_index("c")
        base = wid * B_PER_W

        idx_bufs = (idx0, idx1)
        rows_bufs = (rows0, rows1)
        sems_i = (si0, si1)
        sems_g = (sg0, sg1)
        sems_s = (ss0, ss1)

        def issue_idx(c, b):
            off = pl.multiple_of(base + c * CHUNK, CHUNK)
            pltpu.async_copy(
                idx_hbm.at[pl.ds(off, CHUNK)], idx_bufs[b], sems_i[b])

        def wait_idx(b):
            pltpu.make_async_copy(
                idx_hbm.at[pl.ds(0, CHUNK)], idx_bufs[b], sems_i[b]).wait()

        def issue_gathers(b):
            pltpu.async_copy(
                table_hbm.at[idx_bufs[b]], rows_bufs[b], sems_g[b])

        def wait_gathers(b):
            pltpu.make_async_copy(
                out_hbm.at[pl.ds(0, CHUNK)], rows_bufs[b], sems_g[b]).wait()

        def issue_store(c, b):
            off = pl.multiple_of(base + c * CHUNK, CHUNK)
            pltpu.async_copy(
                rows_bufs[b], out_hbm.at[pl.ds(off, CHUNK)], sems_s[b])

        def wait_store(b):
            pltpu.make_async_copy(
                rows_bufs[b], out_hbm.at[pl.ds(0, CHUNK)], sems_s[b]).wait()

        issue_idx(0, 0)
        issue_idx(1, 1)
        wait_idx(0)
        issue_gathers(0)

        wait_gathers(0)
        issue_idx(2, 0)
        issue_store(0, 0)
        wait_idx(1)
        issue_gathers(1)
        wait_gathers(1)
        issue_idx(3, 1)
        issue_store(1, 1)
        wait_store(0)
        wait_idx(0)
        issue_gathers(0)

        def body(i, carry):
            g = 2 * i
            wait_gathers(0)
            issue_idx(g + 2, 0)
            issue_store(g, 0)
            wait_store(1)
            wait_idx(1)
            issue_gathers(1)
            wait_gathers(1)
            issue_idx(g + 3, 1)
            issue_store(g + 1, 1)
            wait_store(0)
            wait_idx(0)
            issue_gathers(0)
            return carry

        lax.fori_loop(1, N_CHUNKS // 2 - 1, body, 0)

        wait_gathers(0)
        issue_store(N_CHUNKS - 2, 0)
        wait_store(1)
        wait_idx(1)
        issue_gathers(1)
        wait_gathers(1)
        issue_store(N_CHUNKS - 1, 1)
        wait_store(0)
        wait_store(1)

    return gather_kernel


_GATHER = _make_kernel()


def kernel(input_tokens, table):
    idx = input_tokens.reshape(B).astype(jnp.int32)
    out = _GATHER(idx, table)
    return out.reshape(BATCH, HIST, EMBED_DIM)
